# Initial kernel scaffold; baseline (speedup 1.0000x reference)
import jax
import jax.numpy as jnp
from jax.experimental import pallas as pl


def kernel(x, table, W0, W1, W2):
    raise NotImplementedError("write your pallas kernel here")


# speedup vs baseline: 195.4978x; 195.4978x over previous
import dataclasses
import functools

import jax
import jax.numpy as jnp
from jax import lax
from jax.experimental import pallas as pl
from jax.experimental.pallas import tpu as pltpu
from jax.experimental.pallas import tpu_sc as plsc

NUM_LEVELS = 8
LEVEL_DIM = 2
_BASE_RES = 8
_DESIRED_RES = 32
_LOG2_HASH = 18


def _level_meta():
    import math
    s = math.log2(_DESIRED_RES / _BASE_RES) / (NUM_LEVELS - 1)
    metas, off = [], 0
    for l in range(NUM_LEVELS):
        scale = _BASE_RES * (2.0 ** (l * s)) - 1.0
        res = int(math.ceil(scale)) + 1
        size = min(2 ** _LOG2_HASH, (res + 1) ** 3)
        size = int(math.ceil(size / 8) * 8)
        metas.append((float(scale), res, off))
        off += size
    return metas, off


_LEVELS, _TOTAL_ROWS = _level_meta()

_NW = 32
_L = 16


def _encode_body(xh_hbm, yh_hbm, zh_hbm, words_hbm, h_hbm, xs_v, ys_v, zs_v,
                 h_v, table_v, n_points, block_p):
    wid = lax.axis_index("s") * 2 + lax.axis_index("c")
    per_tile = n_points // _NW
    n_blocks = per_tile // block_p
    pltpu.sync_copy(words_hbm, table_v)
    lane = lax.iota(jnp.int32, _L)
    lane16 = lane * NUM_LEVELS * LEVEL_DIM

    @pl.loop(0, n_blocks)
    def _block(b):
        base = wid * per_tile + b * block_p
        pltpu.sync_copy(xh_hbm.at[pl.ds(base, block_p)], xs_v)
        pltpu.sync_copy(yh_hbm.at[pl.ds(base, block_p)], ys_v)
        pltpu.sync_copy(zh_hbm.at[pl.ds(base, block_p)], zs_v)

        @pl.loop(0, block_p // _L)
        def _group(g):
            s = g * _L
            xs = xs_v[pl.ds(s, _L)]
            ys = ys_v[pl.ds(s, _L)]
            zs = zs_v[pl.ds(s, _L)]
            col_base = s * (NUM_LEVELS * LEVEL_DIM)
            for l, (scale, res, off) in enumerate(_LEVELS):
                px = xs * scale + 0.5
                py = ys * scale + 0.5
                pz = zs * scale + 0.5
                ix = px.astype(jnp.int32)
                iy = py.astype(jnp.int32)
                iz = pz.astype(jnp.int32)
                fx = px - ix.astype(jnp.float32)
                fy = py - iy.astype(jnp.float32)
                fz = pz - iz.astype(jnp.float32)
                base_idx = ix + iy * res + iz * (res * res) + off
                wx = (1.0 - fx, fx)
                wy = (1.0 - fy, fy)
                wz = (1.0 - fz, fz)
                acc0 = jnp.zeros((_L,), jnp.float32)
                acc1 = jnp.zeros((_L,), jnp.float32)
                for c in range(8):
                    cx, cy, cz = c & 1, (c >> 1) & 1, (c >> 2) & 1
                    coff = cx + cy * res + cz * (res * res)
                    word = plsc.load_gather(table_v, [base_idx + coff])
                    f0 = plsc.bitcast(word << 16, jnp.float32)
                    f1 = plsc.bitcast(word & jnp.int32(-65536), jnp.float32)
                    w = wx[cx] * wy[cy] * wz[cz]
                    acc0 = acc0 + w * f0
                    acc1 = acc1 + w * f1
                col = lane16 + (col_base + 2 * l)
                plsc.store_scatter(h_v, [col], acc0)
                plsc.store_scatter(h_v, [col + 1], acc1)

        pltpu.sync_copy(
            h_v, h_hbm.at[pl.ds(base * (NUM_LEVELS * LEVEL_DIM),
                                block_p * NUM_LEVELS * LEVEL_DIM)])


def _sc_encode(xh, yh, zh, words, n_points, block_p=1024):
    feat = NUM_LEVELS * LEVEL_DIM
    mesh = plsc.VectorSubcoreMesh(core_axis_name="c", subcore_axis_name="s")
    body = functools.partial(_encode_body, n_points=n_points, block_p=block_p)
    cp = pltpu.CompilerParams()
    if "needs_layout_passes" in pltpu.CompilerParams.__dataclass_fields__:
        cp = dataclasses.replace(cp, needs_layout_passes=False)
    k = pl.kernel(
        body,
        compiler_params=cp,
        out_type=jax.ShapeDtypeStruct((n_points * feat,), jnp.float32),
        mesh=mesh,
        scratch_types=[
            pltpu.VMEM((block_p,), jnp.float32),
            pltpu.VMEM((block_p,), jnp.float32),
            pltpu.VMEM((block_p,), jnp.float32),
            pltpu.VMEM((block_p * feat,), jnp.float32),
            pltpu.VMEM((_TOTAL_ROWS,), jnp.int32),
        ],
    )
    return k(xh, yh, zh, words)


def _mlp_body(h_ref, w0_ref, w1_ref, w2_ref, o_ref):
    h = h_ref[...]
    a = jnp.maximum(jnp.dot(h, w0_ref[...], preferred_element_type=jnp.float32), 0.0)
    a = jnp.maximum(jnp.dot(a, w1_ref[...], preferred_element_type=jnp.float32), 0.0)
    o_ref[...] = jnp.dot(a, w2_ref[...], preferred_element_type=jnp.float32)


def _tc_mlp(h, W0, W1, W2, block_n=16384):
    n = h.shape[0]
    grid = (n // block_n,)
    return pl.pallas_call(
        _mlp_body,
        grid=grid,
        in_specs=[
            pl.BlockSpec((block_n, 16), lambda i: (i, 0)),
            pl.BlockSpec((16, 16), lambda i: (0, 0)),
            pl.BlockSpec((16, 16), lambda i: (0, 0)),
            pl.BlockSpec((16, 1), lambda i: (0, 0)),
        ],
        out_specs=pl.BlockSpec((block_n, 1), lambda i: (i, 0)),
        out_shape=jax.ShapeDtypeStruct((n, 1), jnp.float32),
    )(h, W0, W1, W2)


def kernel(x, table, W0, W1, W2):
    n = x.shape[0]
    t16 = table.astype(jnp.bfloat16)
    bits = lax.bitcast_convert_type(t16, jnp.uint16).astype(jnp.uint32)
    words = lax.bitcast_convert_type(bits[:, 0] | (bits[:, 1] << 16), jnp.int32)
    xh, yh, zh = x[:, 0], x[:, 1], x[:, 2]

    h_flat = _sc_encode(xh, yh, zh, words, n)
    h = h_flat.reshape(n, NUM_LEVELS * LEVEL_DIM)
    return _tc_mlp(h, W0, W1, W2)


# speedup vs baseline: 428.9325x; 2.1941x over previous
import dataclasses
import functools

import jax
import jax.numpy as jnp
from jax import lax
from jax.experimental import pallas as pl
from jax.experimental.pallas import tpu as pltpu
from jax.experimental.pallas import tpu_sc as plsc

NUM_LEVELS = 8
LEVEL_DIM = 2
_BASE_RES = 8
_DESIRED_RES = 32
_LOG2_HASH = 18


def _level_meta():
    import math
    s = math.log2(_DESIRED_RES / _BASE_RES) / (NUM_LEVELS - 1)
    metas, off = [], 0
    for l in range(NUM_LEVELS):
        scale = _BASE_RES * (2.0 ** (l * s)) - 1.0
        res = int(math.ceil(scale)) + 1
        size = min(2 ** _LOG2_HASH, (res + 1) ** 3)
        size = int(math.ceil(size / 8) * 8)
        metas.append((float(scale), res, off))
        off += size
    return metas, off


_LEVELS, _TOTAL_ROWS = _level_meta()

_NW = 32
_L = 16


def _encode_body(xh_hbm, yh_hbm, zh_hbm, words_hbm, h_hbm, xs_v, ys_v, zs_v,
                 h_v, table_v, n_points, block_p):
    wid = lax.axis_index("s") * 2 + lax.axis_index("c")
    per_tile = n_points // _NW
    n_blocks = per_tile // block_p
    pltpu.sync_copy(words_hbm, table_v)
    lane = lax.iota(jnp.int32, _L)

    @pl.loop(0, n_blocks)
    def _block(b):
        base = wid * per_tile + b * block_p
        pltpu.sync_copy(xh_hbm.at[pl.ds(base, block_p)], xs_v)
        pltpu.sync_copy(yh_hbm.at[pl.ds(base, block_p)], ys_v)
        pltpu.sync_copy(zh_hbm.at[pl.ds(base, block_p)], zs_v)

        @pl.loop(0, block_p // _L)
        def _group(g):
            s = g * _L
            xs = xs_v[pl.ds(s, _L)]
            ys = ys_v[pl.ds(s, _L)]
            zs = zs_v[pl.ds(s, _L)]
            p = lane + s
            row = lax.shift_right_logical(p, 3)
            col_base = lax.shift_left(p & 7, 4)
            for l, (scale, res, off) in enumerate(_LEVELS):
                px = xs * scale + 0.5
                py = ys * scale + 0.5
                pz = zs * scale + 0.5
                ix = px.astype(jnp.int32)
                iy = py.astype(jnp.int32)
                iz = pz.astype(jnp.int32)
                fx = px - ix.astype(jnp.float32)
                fy = py - iy.astype(jnp.float32)
                fz = pz - iz.astype(jnp.float32)
                base_idx = ix + iy * res + iz * (res * res) + off
                wx = (1.0 - fx, fx)
                wy = (1.0 - fy, fy)
                wz = (1.0 - fz, fz)
                acc0 = jnp.zeros((_L,), jnp.float32)
                acc1 = jnp.zeros((_L,), jnp.float32)
                for c in range(8):
                    cx, cy, cz = c & 1, (c >> 1) & 1, (c >> 2) & 1
                    coff = cx + cy * res + cz * (res * res)
                    word = plsc.load_gather(table_v, [base_idx + coff])
                    f0 = plsc.bitcast(word << 16, jnp.float32)
                    f1 = plsc.bitcast(word & jnp.int32(-65536), jnp.float32)
                    w = wx[cx] * wy[cy] * wz[cz]
                    acc0 = acc0 + w * f0
                    acc1 = acc1 + w * f1
                col = col_base + 2 * l
                plsc.store_scatter(h_v, [row, col], acc0)
                plsc.store_scatter(h_v, [row, col + 1], acc1)

        row0 = pl.multiple_of(base // 8, 8)
        pltpu.sync_copy(h_v, h_hbm.at[pl.ds(row0, block_p // 8)])


def _sc_encode(xh, yh, zh, words, n_points, block_p=1024):
    feat = NUM_LEVELS * LEVEL_DIM
    mesh = plsc.VectorSubcoreMesh(core_axis_name="c", subcore_axis_name="s")
    body = functools.partial(_encode_body, n_points=n_points, block_p=block_p)
    cp = pltpu.CompilerParams()
    if "needs_layout_passes" in pltpu.CompilerParams.__dataclass_fields__:
        cp = dataclasses.replace(cp, needs_layout_passes=False)
    k = pl.kernel(
        body,
        compiler_params=cp,
        out_type=jax.ShapeDtypeStruct((n_points * feat // 128, 128),
                                      jnp.float32),
        mesh=mesh,
        scratch_types=[
            pltpu.VMEM((block_p,), jnp.float32),
            pltpu.VMEM((block_p,), jnp.float32),
            pltpu.VMEM((block_p,), jnp.float32),
            pltpu.VMEM((block_p * feat // 128, 128), jnp.float32),
            pltpu.VMEM((_TOTAL_ROWS,), jnp.int32),
        ],
    )
    return k(xh, yh, zh, words)


def _mlp_body(h_ref, w0_ref, w1_ref, w2_ref, o_ref):
    h = h_ref[...]
    a = jnp.maximum(jnp.dot(h, w0_ref[...], preferred_element_type=jnp.float32), 0.0)
    a = jnp.maximum(jnp.dot(a, w1_ref[...], preferred_element_type=jnp.float32), 0.0)
    o_ref[...] = jnp.dot(a, w2_ref[...], preferred_element_type=jnp.float32)


def _tc_mlp(h8, W0b, W1b, W2b, block_r=4096):
    rows = h8.shape[0]
    grid = (rows // block_r,)
    return pl.pallas_call(
        _mlp_body,
        grid=grid,
        in_specs=[
            pl.BlockSpec((block_r, 128), lambda i: (i, 0)),
            pl.BlockSpec((128, 128), lambda i: (0, 0)),
            pl.BlockSpec((128, 128), lambda i: (0, 0)),
            pl.BlockSpec((128, 8), lambda i: (0, 0)),
        ],
        out_specs=pl.BlockSpec((block_r, 8), lambda i: (i, 0)),
        out_shape=jax.ShapeDtypeStruct((rows, 8), jnp.float32),
    )(h8, W0b, W1b, W2b)


def kernel(x, table, W0, W1, W2):
    n = x.shape[0]
    t16 = table.astype(jnp.bfloat16)
    bits = lax.bitcast_convert_type(t16, jnp.uint16).astype(jnp.uint32)
    words = lax.bitcast_convert_type(bits[:, 0] | (bits[:, 1] << 16), jnp.int32)
    xh, yh, zh = x[:, 0], x[:, 1], x[:, 2]
    eye8 = jnp.eye(8, dtype=jnp.float32)
    W0b = jnp.kron(eye8, W0)
    W1b = jnp.kron(eye8, W1)
    W2b = jnp.kron(eye8, W2)

    h8 = _sc_encode(xh, yh, zh, words, n)
    o8 = _tc_mlp(h8, W0b, W1b, W2b)
    return o8.reshape(n, 1)


# speedup vs baseline: 470.2069x; 1.0962x over previous
import dataclasses
import functools

import jax
import jax.numpy as jnp
from jax import lax
from jax.experimental import pallas as pl
from jax.experimental.pallas import tpu as pltpu
from jax.experimental.pallas import tpu_sc as plsc

NUM_LEVELS = 8
LEVEL_DIM = 2
_BASE_RES = 8
_DESIRED_RES = 32
_LOG2_HASH = 18


def _level_meta():
    import math
    s = math.log2(_DESIRED_RES / _BASE_RES) / (NUM_LEVELS - 1)
    metas, off = [], 0
    for l in range(NUM_LEVELS):
        scale = _BASE_RES * (2.0 ** (l * s)) - 1.0
        res = int(math.ceil(scale)) + 1
        size = min(2 ** _LOG2_HASH, (res + 1) ** 3)
        size = int(math.ceil(size / 8) * 8)
        metas.append((float(scale), res, off))
        off += size
    return metas, off


_LEVELS, _TOTAL_ROWS = _level_meta()

_NW = 32
_L = 16


def _encode_body(xh_hbm, yh_hbm, zh_hbm, words_hbm, h_hbm, xs_v, ys_v, zs_v,
                 h_v, table_v, n_points, block_p):
    wid = lax.axis_index("s") * 2 + lax.axis_index("c")
    per_tile = n_points // _NW
    n_blocks = per_tile // block_p
    pltpu.sync_copy(words_hbm, table_v)
    lane = lax.iota(jnp.int32, _L)

    @pl.loop(0, n_blocks)
    def _block(b):
        base = wid * per_tile + b * block_p
        pltpu.sync_copy(xh_hbm.at[pl.ds(base, block_p)], xs_v)
        pltpu.sync_copy(yh_hbm.at[pl.ds(base, block_p)], ys_v)
        pltpu.sync_copy(zh_hbm.at[pl.ds(base, block_p)], zs_v)

        @pl.loop(0, block_p // _L)
        def _group(g):
            s = g * _L
            xs = xs_v[pl.ds(s, _L)]
            ys = ys_v[pl.ds(s, _L)]
            zs = zs_v[pl.ds(s, _L)]
            p = lane + s
            row = lax.shift_right_logical(p, 3)
            col_base = lax.shift_left(p & 7, 4)
            for l, (scale, res, off) in enumerate(_LEVELS):
                px = xs * scale + 0.5
                py = ys * scale + 0.5
                pz = zs * scale + 0.5
                ix = px.astype(jnp.int32)
                iy = py.astype(jnp.int32)
                iz = pz.astype(jnp.int32)
                fx = px - ix.astype(jnp.float32)
                fy = py - iy.astype(jnp.float32)
                fz = pz - iz.astype(jnp.float32)
                base_idx = ix + iy * res + iz * (res * res)
                wx = (1.0 - fx, fx)
                wy = (1.0 - fy, fy)
                wz = (1.0 - fz, fz)
                acc = jnp.zeros((2 * _L,), jnp.bfloat16)
                for c in range(8):
                    cx, cy, cz = c & 1, (c >> 1) & 1, (c >> 2) & 1
                    coff = off + cx + cy * res + cz * (res * res)
                    word = plsc.load_gather(table_v, [base_idx + coff])
                    feats = plsc.bitcast(word, jnp.bfloat16)
                    w = wx[cx] * wy[cy] * wz[cz]
                    w2 = plsc.pack(w, w, format=plsc.PackFormat.INTERLEAVED)
                    acc = acc + w2 * feats
                a0, a1 = plsc.unpack(acc, format=plsc.PackFormat.INTERLEAVED)
                col = col_base + 2 * l
                plsc.store_scatter(h_v, [row, col], a0)
                plsc.store_scatter(h_v, [row, col + 1], a1)

        row0 = pl.multiple_of(base // 8, 8)
        pltpu.sync_copy(h_v, h_hbm.at[pl.ds(row0, block_p // 8)])


def _sc_encode(xh, yh, zh, words, n_points, block_p=1024):
    feat = NUM_LEVELS * LEVEL_DIM
    mesh = plsc.VectorSubcoreMesh(core_axis_name="c", subcore_axis_name="s")
    body = functools.partial(_encode_body, n_points=n_points, block_p=block_p)
    cp = pltpu.CompilerParams()
    if "needs_layout_passes" in pltpu.CompilerParams.__dataclass_fields__:
        cp = dataclasses.replace(cp, needs_layout_passes=False)
    k = pl.kernel(
        body,
        compiler_params=cp,
        out_type=jax.ShapeDtypeStruct((n_points * feat // 128, 128),
                                      jnp.float32),
        mesh=mesh,
        scratch_types=[
            pltpu.VMEM((block_p,), jnp.float32),
            pltpu.VMEM((block_p,), jnp.float32),
            pltpu.VMEM((block_p,), jnp.float32),
            pltpu.VMEM((block_p * feat // 128, 128), jnp.float32),
            pltpu.VMEM((_TOTAL_ROWS,), jnp.int32),
        ],
    )
    return k(xh, yh, zh, words)


def _mlp_body(h_ref, w0_ref, w1_ref, w2_ref, o_ref):
    h = h_ref[...]
    a = jnp.maximum(jnp.dot(h, w0_ref[...], preferred_element_type=jnp.float32), 0.0)
    a = jnp.maximum(jnp.dot(a, w1_ref[...], preferred_element_type=jnp.float32), 0.0)
    o_ref[...] = jnp.dot(a, w2_ref[...], preferred_element_type=jnp.float32)


def _tc_mlp(h8, W0b, W1b, W2b, block_r=4096):
    rows = h8.shape[0]
    grid = (rows // block_r,)
    return pl.pallas_call(
        _mlp_body,
        grid=grid,
        in_specs=[
            pl.BlockSpec((block_r, 128), lambda i: (i, 0)),
            pl.BlockSpec((128, 128), lambda i: (0, 0)),
            pl.BlockSpec((128, 128), lambda i: (0, 0)),
            pl.BlockSpec((128, 8), lambda i: (0, 0)),
        ],
        out_specs=pl.BlockSpec((block_r, 8), lambda i: (i, 0)),
        out_shape=jax.ShapeDtypeStruct((rows, 8), jnp.float32),
    )(h8, W0b, W1b, W2b)


def kernel(x, table, W0, W1, W2):
    n = x.shape[0]
    t16 = table.astype(jnp.bfloat16)
    bits = lax.bitcast_convert_type(t16, jnp.uint16).astype(jnp.uint32)
    words = lax.bitcast_convert_type(bits[:, 0] | (bits[:, 1] << 16), jnp.int32)
    xh, yh, zh = x[:, 0], x[:, 1], x[:, 2]
    eye8 = jnp.eye(8, dtype=jnp.float32)
    W0b = jnp.kron(eye8, W0)
    W1b = jnp.kron(eye8, W1)
    W2b = jnp.kron(eye8, W2)

    h8 = _sc_encode(xh, yh, zh, words, n)
    o8 = _tc_mlp(h8, W0b, W1b, W2b)
    return o8.reshape(n, 1)


# speedup vs baseline: 538.7659x; 1.1458x over previous
import dataclasses
import functools

import jax
import jax.numpy as jnp
from jax import lax
from jax.experimental import pallas as pl
from jax.experimental.pallas import tpu as pltpu
from jax.experimental.pallas import tpu_sc as plsc

NUM_LEVELS = 8
LEVEL_DIM = 2
_BASE_RES = 8
_DESIRED_RES = 32
_LOG2_HASH = 18


def _level_meta():
    import math
    s = math.log2(_DESIRED_RES / _BASE_RES) / (NUM_LEVELS - 1)
    metas, off = [], 0
    for l in range(NUM_LEVELS):
        scale = _BASE_RES * (2.0 ** (l * s)) - 1.0
        res = int(math.ceil(scale)) + 1
        size = min(2 ** _LOG2_HASH, (res + 1) ** 3)
        size = int(math.ceil(size / 8) * 8)
        metas.append((float(scale), res, off))
        off += size
    return metas, off


_LEVELS, _TOTAL_ROWS = _level_meta()

_NW = 32
_L = 16


def _encode_body(xh_hbm, yh_hbm, zh_hbm, words_hbm, h_hbm,
                 xs_a, ys_a, zs_a, xs_b, ys_b, zs_b, h_a, h_b, table_v,
                 semx_a, semx_b, semh_a, semh_b, semt,
                 n_points, block_p):
    wid = lax.axis_index("s") * 2 + lax.axis_index("c")
    per_tile = n_points // _NW
    n_blocks = per_tile // block_p
    lane = lax.iota(jnp.int32, _L)

    def start_x(b, xs_v, ys_v, zs_v, sem):
        base = wid * per_tile + b * block_p
        pltpu.async_copy(xh_hbm.at[pl.ds(base, block_p)], xs_v, sem)
        pltpu.async_copy(yh_hbm.at[pl.ds(base, block_p)], ys_v, sem)
        pltpu.async_copy(zh_hbm.at[pl.ds(base, block_p)], zs_v, sem)

    def wait_x(xs_v, ys_v, zs_v, sem):
        pltpu.make_async_copy(xh_hbm.at[pl.ds(0, block_p)], xs_v, sem).wait()
        pltpu.make_async_copy(yh_hbm.at[pl.ds(0, block_p)], ys_v, sem).wait()
        pltpu.make_async_copy(zh_hbm.at[pl.ds(0, block_p)], zs_v, sem).wait()

    def start_h(b, h_v, sem):
        base = wid * per_tile + b * block_p
        row0 = pl.multiple_of(base // 8, 8)
        pltpu.async_copy(h_v, h_hbm.at[pl.ds(row0, block_p // 8)], sem)

    def wait_h(h_v, sem):
        pltpu.make_async_copy(
            h_v, h_hbm.at[pl.ds(0, block_p // 8)], sem).wait()

    def compute(b, xs_v, ys_v, zs_v, h_v):
        @pl.loop(0, block_p // _L)
        def _group(g):
            s = g * _L
            xs = xs_v[pl.ds(s, _L)]
            ys = ys_v[pl.ds(s, _L)]
            zs = zs_v[pl.ds(s, _L)]
            p = lane + s
            row = lax.shift_right_logical(p, 3)
            col_base = lax.shift_left(p & 7, 4)
            for l, (scale, res, off) in enumerate(_LEVELS):
                px = xs * scale + 0.5
                py = ys * scale + 0.5
                pz = zs * scale + 0.5
                ix = px.astype(jnp.int32)
                iy = py.astype(jnp.int32)
                iz = pz.astype(jnp.int32)
                fx = px - ix.astype(jnp.float32)
                fy = py - iy.astype(jnp.float32)
                fz = pz - iz.astype(jnp.float32)
                base_idx = ix + iy * res + iz * (res * res)
                wx = (1.0 - fx, fx)
                wy = (1.0 - fy, fy)
                wz = (1.0 - fz, fz)
                acc = jnp.zeros((2 * _L,), jnp.bfloat16)
                for c in range(8):
                    cx, cy, cz = c & 1, (c >> 1) & 1, (c >> 2) & 1
                    coff = off + cx + cy * res + cz * (res * res)
                    word = plsc.load_gather(table_v, [base_idx + coff])
                    feats = plsc.bitcast(word, jnp.bfloat16)
                    w = wx[cx] * wy[cy] * wz[cz]
                    w2 = plsc.pack(w, w, format=plsc.PackFormat.INTERLEAVED)
                    acc = acc + w2 * feats
                a0, a1 = plsc.unpack(acc, format=plsc.PackFormat.INTERLEAVED)
                col = col_base + 2 * l
                plsc.store_scatter(h_v, [row, col], a0)
                plsc.store_scatter(h_v, [row, col + 1], a1)

    pltpu.async_copy(words_hbm, table_v, semt)
    start_x(0, xs_a, ys_a, zs_a, semx_a)
    pltpu.make_async_copy(words_hbm, table_v, semt).wait()

    @pl.loop(0, n_blocks, step=2)
    def _block(b):
        wait_x(xs_a, ys_a, zs_a, semx_a)
        start_x(b + 1, xs_b, ys_b, zs_b, semx_b)

        @pl.when(b > 0)
        def _():
            wait_h(h_a, semh_a)
        compute(b, xs_a, ys_a, zs_a, h_a)
        start_h(b, h_a, semh_a)

        wait_x(xs_b, ys_b, zs_b, semx_b)

        @pl.when(b + 2 < n_blocks)
        def _():
            start_x(b + 2, xs_a, ys_a, zs_a, semx_a)

        @pl.when(b > 0)
        def _():
            wait_h(h_b, semh_b)
        compute(b + 1, xs_b, ys_b, zs_b, h_b)
        start_h(b + 1, h_b, semh_b)

    wait_h(h_a, semh_a)
    wait_h(h_b, semh_b)


def _sc_encode(xh, yh, zh, words, n_points, block_p=512):
    feat = NUM_LEVELS * LEVEL_DIM
    mesh = plsc.VectorSubcoreMesh(core_axis_name="c", subcore_axis_name="s")
    body = functools.partial(_encode_body, n_points=n_points, block_p=block_p)
    cp = pltpu.CompilerParams()
    if "needs_layout_passes" in pltpu.CompilerParams.__dataclass_fields__:
        cp = dataclasses.replace(cp, needs_layout_passes=False)
    k = pl.kernel(
        body,
        compiler_params=cp,
        out_type=jax.ShapeDtypeStruct((n_points * feat // 128, 128),
                                      jnp.float32),
        mesh=mesh,
        scratch_types=[
            pltpu.VMEM((block_p,), jnp.float32),
            pltpu.VMEM((block_p,), jnp.float32),
            pltpu.VMEM((block_p,), jnp.float32),
            pltpu.VMEM((block_p,), jnp.float32),
            pltpu.VMEM((block_p,), jnp.float32),
            pltpu.VMEM((block_p,), jnp.float32),
            pltpu.VMEM((block_p * feat // 128, 128), jnp.float32),
            pltpu.VMEM((block_p * feat // 128, 128), jnp.float32),
            pltpu.VMEM((_TOTAL_ROWS,), jnp.int32),
            pltpu.SemaphoreType.DMA,
            pltpu.SemaphoreType.DMA,
            pltpu.SemaphoreType.DMA,
            pltpu.SemaphoreType.DMA,
            pltpu.SemaphoreType.DMA,
        ],
    )
    return k(xh, yh, zh, words)


def _mlp_body(h_ref, w0_ref, w1_ref, w2_ref, o_ref):
    h = h_ref[...]
    a = jnp.maximum(jnp.dot(h, w0_ref[...], preferred_element_type=jnp.float32), 0.0)
    a = jnp.maximum(jnp.dot(a, w1_ref[...], preferred_element_type=jnp.float32), 0.0)
    o_ref[...] = jnp.dot(a, w2_ref[...], preferred_element_type=jnp.float32)


def _tc_mlp(h8, W0b, W1b, W2b, block_r=4096):
    rows = h8.shape[0]
    grid = (rows // block_r,)
    return pl.pallas_call(
        _mlp_body,
        grid=grid,
        in_specs=[
            pl.BlockSpec((block_r, 128), lambda i: (i, 0)),
            pl.BlockSpec((128, 128), lambda i: (0, 0)),
            pl.BlockSpec((128, 128), lambda i: (0, 0)),
            pl.BlockSpec((128, 8), lambda i: (0, 0)),
        ],
        out_specs=pl.BlockSpec((block_r, 8), lambda i: (i, 0)),
        out_shape=jax.ShapeDtypeStruct((rows, 8), jnp.float32),
    )(h8, W0b, W1b, W2b)


def kernel(x, table, W0, W1, W2):
    n = x.shape[0]
    t16 = table.astype(jnp.bfloat16)
    bits = lax.bitcast_convert_type(t16, jnp.uint16).astype(jnp.uint32)
    words = lax.bitcast_convert_type(bits[:, 0] | (bits[:, 1] << 16), jnp.int32)
    xh, yh, zh = x[:, 0], x[:, 1], x[:, 2]
    eye8 = jnp.eye(8, dtype=jnp.float32)
    W0b = jnp.kron(eye8, W0)
    W1b = jnp.kron(eye8, W1)
    W2b = jnp.kron(eye8, W2)

    h8 = _sc_encode(xh, yh, zh, words, n)
    o8 = _tc_mlp(h8, W0b, W1b, W2b)
    return o8.reshape(n, 1)
